# baseline (speedup 1.0000x reference)
import functools

import jax
import jax.numpy as jnp
from jax.experimental import pallas as pl
from jax.experimental.pallas import tpu as pltpu

K_NN = 5
N_CLASSES = 10
NEG_INF = -1e30
LANES = 128


def _knn_kernel(n_chunks, xq_ref, xt_ref, lbl_ref, out_ref, g_refs):
    i = pl.program_id(0)
    chunk = xt_ref.shape[0]
    ntiles = chunk // LANES
    q = xq_ref.shape[0]

    xq_aug = xq_ref[...] * 2.0

    first = i == 0
    g = [jnp.where(first, NEG_INF, r[...]) for r in g_refs]

    for j in range(ntiles):
        xc = xt_ref[j * LANES:(j + 1) * LANES, :]
        lbl = lbl_ref[:, j * LANES:(j + 1) * LANES]
        ksq = jnp.sum(xc * xc, axis=1)[None, :]
        s = jax.lax.dot_general(
            xq_aug, xc, (((1,), (1,)), ((), ())),
            preferred_element_type=jnp.float32) - ksq
        x = jax.lax.bitcast_convert_type(
            (jax.lax.bitcast_convert_type(s, jnp.int32) & jnp.int32(-16))
            | lbl, jnp.float32)
        c = 3 * (j % 2)
        g1, g2, g3 = g[c], g[c + 1], g[c + 2]
        t1 = jnp.minimum(g1, x)
        g[c] = jnp.maximum(g1, x)
        t2 = jnp.minimum(g2, t1)
        g[c + 1] = jnp.maximum(g2, t1)
        g[c + 2] = jnp.maximum(g3, t2)

    for r, v in zip(g_refs, g):
        r[...] = v

    @pl.when(i == n_chunks - 1)
    def _emit():
        comb = jnp.concatenate(g, axis=1)
        best = []
        for _ in range(K_NN):
            m = jnp.max(comb, axis=1, keepdims=True)
            best.append(m)
            comb = jnp.where(comb == m, NEG_INF, comb)
        bs = jnp.concatenate(best, axis=1)
        bl = jax.lax.bitcast_convert_type(bs, jnp.int32) & 15
        mx = jnp.max(bs, axis=1, keepdims=True)
        e = jnp.exp(bs - mx)
        w = e / jnp.sum(e, axis=1, keepdims=True)
        cls = jax.lax.broadcasted_iota(jnp.int32, (1, N_CLASSES), 1)
        onehot = (bl[:, :, None] == cls[None, :, :]).astype(jnp.float32)
        out_ref[...] = jnp.sum(w[:, :, None] * onehot, axis=1)


def kernel(x_test, x_train, y_train):
    q, d = x_test.shape
    n_train = x_train.shape[0]
    chunk = 2048
    n_chunks = (n_train + chunk - 1) // chunk
    n_pad = n_chunks * chunk
    pad_row = jnp.zeros((n_pad - n_train, d), jnp.float32
                        ).at[:, 0].set(1e4)
    xt = jnp.concatenate([x_train, pad_row], axis=0)
    lbl = jnp.pad(y_train, (0, n_pad - n_train))[None, :]

    grid = (n_chunks,)
    f = lambda *refs: _knn_kernel(n_chunks, *refs[:4], list(refs[4:]))
    return pl.pallas_call(
        f,
        grid=grid,
        in_specs=[
            pl.BlockSpec((q, d), lambda i: (0, 0)),
            pl.BlockSpec((chunk, d), lambda i: (i, 0)),
            pl.BlockSpec((1, chunk), lambda i: (0, i)),
        ],
        out_specs=pl.BlockSpec((q, N_CLASSES), lambda i: (0, 0)),
        out_shape=jax.ShapeDtypeStruct((q, N_CLASSES), jnp.float32),
        scratch_shapes=[pltpu.VMEM((q, LANES), jnp.float32)
                        for _ in range(6)],
    )(x_test, xt, lbl)


# speedup vs baseline: 1.0252x; 1.0252x over previous
import functools

import jax
import jax.numpy as jnp
from jax.experimental import pallas as pl
from jax.experimental.pallas import tpu as pltpu

K_NN = 5
N_CLASSES = 10
NEG_INF = -1e30
LANES = 128


def _knn_kernel(n_chunks, xq_ref, xt_ref, lbl_ref, out_ref, g_refs):
    i = pl.program_id(0)
    chunk = xt_ref.shape[0]
    ntiles = chunk // LANES
    q = xq_ref.shape[0]

    xq_aug = xq_ref[...] * 2.0

    @pl.when(i == 0)
    def _init():
        for r in g_refs:
            r[...] = jnp.full(r.shape, NEG_INF, jnp.float32)

    g = [r[...] for r in g_refs]

    for j in range(ntiles):
        xc = xt_ref[j * LANES:(j + 1) * LANES, :]
        lbl = lbl_ref[:, j * LANES:(j + 1) * LANES]
        ksq = jnp.sum(xc * xc, axis=1)[None, :]
        s = jax.lax.dot_general(
            xq_aug, xc, (((1,), (1,)), ((), ())),
            preferred_element_type=jnp.float32) - ksq
        x = jax.lax.bitcast_convert_type(
            (jax.lax.bitcast_convert_type(s, jnp.int32) & jnp.int32(-16))
            | lbl, jnp.float32)
        c = 3 * (j % 2)
        g1, g2, g3 = g[c], g[c + 1], g[c + 2]
        t1 = jnp.minimum(g1, x)
        g[c] = jnp.maximum(g1, x)
        t2 = jnp.minimum(g2, t1)
        g[c + 1] = jnp.maximum(g2, t1)
        g[c + 2] = jnp.maximum(g3, t2)

    for r, v in zip(g_refs, g):
        r[...] = v

    @pl.when(i == n_chunks - 1)
    def _emit():
        comb = jnp.concatenate(g, axis=1)
        best = []
        for _ in range(K_NN):
            m = jnp.max(comb, axis=1, keepdims=True)
            best.append(m)
            comb = jnp.where(comb == m, NEG_INF, comb)
        bs = jnp.concatenate(best, axis=1)
        bl = jax.lax.bitcast_convert_type(bs, jnp.int32) & 15
        mx = jnp.max(bs, axis=1, keepdims=True)
        e = jnp.exp(bs - mx)
        w = e / jnp.sum(e, axis=1, keepdims=True)
        cls = jax.lax.broadcasted_iota(jnp.int32, (1, N_CLASSES), 1)
        onehot = (bl[:, :, None] == cls[None, :, :]).astype(jnp.float32)
        out_ref[...] = jnp.sum(w[:, :, None] * onehot, axis=1)


def kernel(x_test, x_train, y_train):
    q, d = x_test.shape
    n_train = x_train.shape[0]
    chunk = 4096
    n_chunks = (n_train + chunk - 1) // chunk
    n_pad = n_chunks * chunk
    pad_row = jnp.zeros((n_pad - n_train, d), jnp.float32
                        ).at[:, 0].set(1e4)
    xt = jnp.concatenate([x_train, pad_row], axis=0)
    lbl = jnp.pad(y_train, (0, n_pad - n_train))[None, :]

    grid = (n_chunks,)
    f = lambda *refs: _knn_kernel(n_chunks, *refs[:4], list(refs[4:]))
    return pl.pallas_call(
        f,
        grid=grid,
        in_specs=[
            pl.BlockSpec((q, d), lambda i: (0, 0)),
            pl.BlockSpec((chunk, d), lambda i: (i, 0)),
            pl.BlockSpec((1, chunk), lambda i: (0, i)),
        ],
        out_specs=pl.BlockSpec((q, N_CLASSES), lambda i: (0, 0)),
        out_shape=jax.ShapeDtypeStruct((q, N_CLASSES), jnp.float32),
        scratch_shapes=[pltpu.VMEM((q, LANES), jnp.float32)
                        for _ in range(6)],
    )(x_test, xt, lbl)


# speedup vs baseline: 1.1674x; 1.1387x over previous
import functools

import jax
import jax.numpy as jnp
from jax.experimental import pallas as pl
from jax.experimental.pallas import tpu as pltpu

K_NN = 5
N_CLASSES = 10
NEG_INF = -1e30
LANES = 128


def _knn_kernel(n_chunks, xq_ref, xt_ref, lbl_ref, out_ref, g_refs):
    i = pl.program_id(0)
    chunk = xt_ref.shape[0]
    ntiles = chunk // LANES
    q = xq_ref.shape[0]

    xq_aug = xq_ref[...] * 2.0

    @pl.when(i == 0)
    def _init():
        for r in g_refs:
            r[...] = jnp.full(r.shape, NEG_INF, jnp.float32)

    g = [r[...] for r in g_refs]

    def packed_tile(j):
        xc = xt_ref[j * LANES:(j + 1) * LANES, :]
        lbl = lbl_ref[:, j * LANES:(j + 1) * LANES]
        ksq = jnp.sum(xc * xc, axis=1)[None, :]
        s = jax.lax.dot_general(
            xq_aug, xc, (((1,), (1,)), ((), ())),
            preferred_element_type=jnp.float32) - ksq
        return jax.lax.bitcast_convert_type(
            (jax.lax.bitcast_convert_type(s, jnp.int32) & jnp.int32(-16))
            | lbl, jnp.float32)

    for j in range(0, ntiles, 4):
        for c, (ja, jb) in ((0, (j, j + 2)), (3, (j + 1, j + 3))):
            x, y = packed_tile(ja), packed_tile(jb)
            hi = jnp.maximum(x, y)
            lo = jnp.minimum(x, y)
            g1, g2, g3 = g[c], g[c + 1], g[c + 2]
            a = jnp.minimum(g1, hi)
            g[c] = jnp.maximum(g1, hi)
            mx2 = jnp.maximum(g2, lo)
            g[c + 1] = jnp.maximum(a, mx2)
            g[c + 2] = jnp.maximum(jnp.minimum(a, mx2), g3)

    for r, v in zip(g_refs, g):
        r[...] = v

    @pl.when(i == n_chunks - 1)
    def _emit():
        comb = jnp.concatenate(g, axis=1)
        best = []
        for _ in range(K_NN):
            m = jnp.max(comb, axis=1, keepdims=True)
            best.append(m)
            comb = jnp.where(comb == m, NEG_INF, comb)
        bs = jnp.concatenate(best, axis=1)
        bl = jax.lax.bitcast_convert_type(bs, jnp.int32) & 15
        mx = jnp.max(bs, axis=1, keepdims=True)
        e = jnp.exp(bs - mx)
        w = e / jnp.sum(e, axis=1, keepdims=True)
        cls = jax.lax.broadcasted_iota(jnp.int32, (1, N_CLASSES), 1)
        onehot = (bl[:, :, None] == cls[None, :, :]).astype(jnp.float32)
        out_ref[...] = jnp.sum(w[:, :, None] * onehot, axis=1)


def kernel(x_test, x_train, y_train):
    q, d = x_test.shape
    n_train = x_train.shape[0]
    chunk = 4096
    n_chunks = (n_train + chunk - 1) // chunk
    n_pad = n_chunks * chunk
    pad_row = jnp.zeros((n_pad - n_train, d), jnp.float32
                        ).at[:, 0].set(1e4)
    xt = jnp.concatenate([x_train, pad_row], axis=0)
    lbl = jnp.pad(y_train, (0, n_pad - n_train))[None, :]

    grid = (n_chunks,)
    f = lambda *refs: _knn_kernel(n_chunks, *refs[:4], list(refs[4:]))
    return pl.pallas_call(
        f,
        grid=grid,
        in_specs=[
            pl.BlockSpec((q, d), lambda i: (0, 0)),
            pl.BlockSpec((chunk, d), lambda i: (i, 0)),
            pl.BlockSpec((1, chunk), lambda i: (0, i)),
        ],
        out_specs=pl.BlockSpec((q, N_CLASSES), lambda i: (0, 0)),
        out_shape=jax.ShapeDtypeStruct((q, N_CLASSES), jnp.float32),
        scratch_shapes=[pltpu.VMEM((q, LANES), jnp.float32)
                        for _ in range(6)],
    )(x_test, xt, lbl)


# speedup vs baseline: 1.2750x; 1.0922x over previous
import functools

import jax
import jax.numpy as jnp
from jax.experimental import pallas as pl
from jax.experimental.pallas import tpu as pltpu

K_NN = 5
N_CLASSES = 10
NEG_INF = -1e30
LANES = 128
AUG = 24


def _knn_kernel(n_chunks, xq_ref, xt_ref, lbl_ref, out_ref,
                xqa_ref, aug_ref, *g_refs):
    i = pl.program_id(0)
    chunk = xt_ref.shape[0]
    ntiles = chunk // LANES
    q = xq_ref.shape[0]
    g_refs = list(g_refs)

    @pl.when(i == 0)
    def _init():
        for r in g_refs:
            r[...] = jnp.full(r.shape, NEG_INF, jnp.float32)
        xqa_ref[...] = jnp.zeros(xqa_ref.shape, jnp.float32)
        xqa_ref[:, 0:16] = xq_ref[...] * 2.0
        xqa_ref[:, 16:17] = jnp.ones((q, 1), jnp.float32)
        aug_ref[...] = jnp.zeros(aug_ref.shape, jnp.float32)

    xc_all = xt_ref[...]
    aug_ref[:, 0:16] = xc_all
    aug_ref[:, 16:17] = -jnp.sum(xc_all * xc_all, axis=1, keepdims=True)
    xqa = xqa_ref[...]

    g = [r[...] for r in g_refs]

    def packed_tile(j):
        xc = aug_ref[j * LANES:(j + 1) * LANES, :]
        lbl = lbl_ref[:, j * LANES:(j + 1) * LANES]
        s = jax.lax.dot_general(
            xqa, xc, (((1,), (1,)), ((), ())),
            preferred_element_type=jnp.float32)
        return jax.lax.bitcast_convert_type(
            (jax.lax.bitcast_convert_type(s, jnp.int32) & jnp.int32(-16))
            | lbl, jnp.float32)

    for j in range(0, ntiles, 4):
        for c, (ja, jb) in ((0, (j, j + 2)), (3, (j + 1, j + 3))):
            x, y = packed_tile(ja), packed_tile(jb)
            hi = jnp.maximum(x, y)
            lo = jnp.minimum(x, y)
            g1, g2, g3 = g[c], g[c + 1], g[c + 2]
            a = jnp.minimum(g1, hi)
            g[c] = jnp.maximum(g1, hi)
            mx2 = jnp.maximum(g2, lo)
            g[c + 1] = jnp.maximum(a, mx2)
            g[c + 2] = jnp.maximum(jnp.minimum(a, mx2), g3)

    for r, v in zip(g_refs, g):
        r[...] = v

    @pl.when(i == n_chunks - 1)
    def _emit():
        comb = jnp.concatenate(g, axis=1)
        best = []
        for _ in range(K_NN):
            m = jnp.max(comb, axis=1, keepdims=True)
            best.append(m)
            comb = jnp.where(comb == m, NEG_INF, comb)
        bs = jnp.concatenate(best, axis=1)
        bl = jax.lax.bitcast_convert_type(bs, jnp.int32) & 15
        mx = jnp.max(bs, axis=1, keepdims=True)
        e = jnp.exp(bs - mx)
        w = e / jnp.sum(e, axis=1, keepdims=True)
        cls = jax.lax.broadcasted_iota(jnp.int32, (1, N_CLASSES), 1)
        onehot = (bl[:, :, None] == cls[None, :, :]).astype(jnp.float32)
        out_ref[...] = jnp.sum(w[:, :, None] * onehot, axis=1)


def kernel(x_test, x_train, y_train):
    q, d = x_test.shape
    n_train = x_train.shape[0]
    chunk = 4096
    n_chunks = (n_train + chunk - 1) // chunk
    n_pad = n_chunks * chunk
    pad_row = jnp.zeros((n_pad - n_train, d), jnp.float32
                        ).at[:, 0].set(1e4)
    xt = jnp.concatenate([x_train, pad_row], axis=0)
    lbl = jnp.pad(y_train, (0, n_pad - n_train))[None, :]

    grid = (n_chunks,)
    return pl.pallas_call(
        functools.partial(_knn_kernel, n_chunks),
        grid=grid,
        in_specs=[
            pl.BlockSpec((q, d), lambda i: (0, 0)),
            pl.BlockSpec((chunk, d), lambda i: (i, 0)),
            pl.BlockSpec((1, chunk), lambda i: (0, i)),
        ],
        out_specs=pl.BlockSpec((q, N_CLASSES), lambda i: (0, 0)),
        out_shape=jax.ShapeDtypeStruct((q, N_CLASSES), jnp.float32),
        scratch_shapes=[pltpu.VMEM((q, AUG), jnp.float32),
                        pltpu.VMEM((chunk, AUG), jnp.float32)]
                       + [pltpu.VMEM((q, LANES), jnp.float32)
                          for _ in range(6)],
    )(x_test, xt, lbl)
